# Initial kernel scaffold; baseline (speedup 1.0000x reference)
import jax
import jax.numpy as jnp
from jax import lax
from jax.experimental import pallas as pl
from jax.experimental.pallas import tpu as pltpu
from jax.experimental.pallas import tpu_sc as plsc

N = 10000
C = 128
E = 320000

NC = 2
NS = 16
B = 128
CHUNKS = -(-E // (NS * B))
EPT = CHUNKS * B
EPAD = NS * EPT
NPAD = 10016
SLICE = NPAD // NS

BLK = 1000
NBLK = N // BLK


def _sc_pass(with_cnt):
    mesh = plsc.VectorSubcoreMesh(
        core_axis_name="c", subcore_axis_name="s", num_cores=NC,
        num_subcores=NS)
    out_type = [jax.ShapeDtypeStruct((NC, NPAD, C), jnp.float32)]
    scratch = [
        pltpu.VMEM((CHUNKS, B), jnp.int32),
        pltpu.VMEM((CHUNKS, B), jnp.int32),
        pltpu.VMEM((B, C), jnp.float32),
        pltpu.SemaphoreType.DMA,
        pltpu.VMEM_SHARED((NPAD, C), jnp.float32),
    ]
    if with_cnt:
        out_type.append(jax.ShapeDtypeStruct((NC, NPAD, 8), jnp.float32))
        scratch += [
            pltpu.VMEM((B, 8), jnp.float32),
            pltpu.VMEM_SHARED((NPAD, 8), jnp.float32),
        ]

    def body_cnt(xcat, srcs, dsts, z_c, z_8, ones_h,
                 s_out, cnt_out, sidx, didx, rows, sem, acc, ones_v, cacc):
        c = lax.axis_index("c")
        s = lax.axis_index("s")
        pltpu.sync_copy(z_c, acc.at[pl.ds(s * SLICE, SLICE)])
        pltpu.sync_copy(z_8, cacc.at[pl.ds(s * SLICE, SLICE)])
        pltpu.sync_copy(ones_h, ones_v)
        pltpu.sync_copy(srcs.at[c, s], sidx)
        pltpu.sync_copy(dsts.at[c, s], didx)
        plsc.subcore_barrier()

        def step(g, carry):
            pltpu.async_copy(xcat.at[sidx.at[g]], rows, sem).wait()
            pltpu.sync_copy(rows, acc.at[didx.at[g]], add=True)
            pltpu.sync_copy(ones_v, cacc.at[didx.at[g]], add=True)
            return carry

        lax.fori_loop(0, CHUNKS, step, 0)
        plsc.subcore_barrier()
        sl = pl.ds(s * SLICE, SLICE)
        pltpu.sync_copy(acc.at[sl], s_out.at[c, sl])
        pltpu.sync_copy(cacc.at[sl], cnt_out.at[c, sl])

    def body_nocnt(xcat, srcs, dsts, z_c,
                   s_out, sidx, didx, rows, sem, acc):
        c = lax.axis_index("c")
        s = lax.axis_index("s")
        pltpu.sync_copy(z_c, acc.at[pl.ds(s * SLICE, SLICE)])
        pltpu.sync_copy(srcs.at[c, s], sidx)
        pltpu.sync_copy(dsts.at[c, s], didx)
        plsc.subcore_barrier()

        def step(g, carry):
            pltpu.async_copy(xcat.at[sidx.at[g]], rows, sem).wait()
            pltpu.sync_copy(rows, acc.at[didx.at[g]], add=True)
            return carry

        lax.fori_loop(0, CHUNKS, step, 0)
        plsc.subcore_barrier()
        sl = pl.ds(s * SLICE, SLICE)
        pltpu.sync_copy(acc.at[sl], s_out.at[c, sl])

    body = body_cnt if with_cnt else body_nocnt
    return pl.kernel(body, out_type=out_type, mesh=mesh,
                     scratch_types=scratch)


def _tc_body(s_ref, cnt_ref, x_ref, wl_ref, bl_ref, wr_ref, lnw_ref,
             lnb_ref, o_ref):
    cnt = cnt_ref[0, :, 0:1]
    mean = s_ref[0] / jnp.maximum(cnt, 1.0)
    h = lax.dot_general(mean, wl_ref[0], (((1,), (1,)), ((), ())),
                        preferred_element_type=jnp.float32)
    h += lax.dot_general(x_ref[...], wr_ref[0], (((1,), (1,)), ((), ())),
                         preferred_element_type=jnp.float32)
    h += bl_ref[0][None, :]
    mu = jnp.mean(h, axis=1, keepdims=True)
    var = jnp.mean((h - mu) * (h - mu), axis=1, keepdims=True)
    y = (h - mu) * lax.rsqrt(var + 1e-5) * lnw_ref[0][None, :] \
        + lnb_ref[0][None, :]
    o_ref[...] = jnp.maximum(y, 0.0)


_tc_pass = pl.pallas_call(
    _tc_body,
    grid=(2, NBLK),
    in_specs=[
        pl.BlockSpec((1, BLK, C), lambda t, i: (1 - t, i, 0)),
        pl.BlockSpec((1, BLK, 8), lambda t, i: (1 - t, i, 0)),
        pl.BlockSpec((BLK, C), lambda t, i: (t * NBLK + i, 0)),
        pl.BlockSpec((1, C, C), lambda t, i: (t, 0, 0)),
        pl.BlockSpec((1, C), lambda t, i: (t, 0)),
        pl.BlockSpec((1, C, C), lambda t, i: (t, 0, 0)),
        pl.BlockSpec((1, C), lambda t, i: (t, 0)),
        pl.BlockSpec((1, C), lambda t, i: (t, 0)),
    ],
    out_specs=pl.BlockSpec((BLK, C), lambda t, i: (t * NBLK + i, 0)),
    out_shape=jax.ShapeDtypeStruct((2 * N, C), jnp.float32),
)


def _prep_edges(ei, src_off):
    pad = EPAD - E
    src = jnp.concatenate(
        [ei[0] + src_off, jnp.zeros((pad,), jnp.int32)])
    dst = jnp.concatenate(
        [ei[1], jnp.full((pad,), N, jnp.int32)])
    return src.reshape(NS, CHUNKS, B), dst.reshape(NS, CHUNKS, B)


def kernel(x_user, x_item, edge_index_user_item, edge_index_item_user,
           l0_ui_Wl, l0_ui_bl, l0_ui_Wr, l0_iu_Wl, l0_iu_bl, l0_iu_Wr,
           l0_ln_user_w, l0_ln_user_b, l0_ln_item_w, l0_ln_item_b,
           l1_ui_Wl, l1_ui_bl, l1_ui_Wr, l1_iu_Wl, l1_iu_bl, l1_iu_Wr,
           l1_ln_user_w, l1_ln_user_b, l1_ln_item_w, l1_ln_item_b):
    su, du = _prep_edges(edge_index_user_item, 0)
    si, di = _prep_edges(edge_index_item_user, N)
    srcs = jnp.stack([su, si])
    dsts = jnp.stack([du, di])
    z_c = jnp.zeros((SLICE, C), jnp.float32)
    z_8 = jnp.zeros((SLICE, 8), jnp.float32)
    ones_h = jnp.ones((B, 8), jnp.float32)

    xcat = jnp.concatenate([x_user, x_item], axis=0)

    layers = [
        (jnp.stack([l0_iu_Wl, l0_ui_Wl]), jnp.stack([l0_iu_bl, l0_ui_bl]),
         jnp.stack([l0_iu_Wr, l0_ui_Wr]),
         jnp.stack([l0_ln_user_w, l0_ln_item_w]),
         jnp.stack([l0_ln_user_b, l0_ln_item_b])),
        (jnp.stack([l1_iu_Wl, l1_ui_Wl]), jnp.stack([l1_iu_bl, l1_ui_bl]),
         jnp.stack([l1_iu_Wr, l1_ui_Wr]),
         jnp.stack([l1_ln_user_w, l1_ln_item_w]),
         jnp.stack([l1_ln_user_b, l1_ln_item_b])),
    ]

    s_all, cnt_all = _sc_pass(True)(xcat, srcs, dsts, z_c, z_8, ones_h)
    xcat = _tc_pass(s_all, cnt_all, xcat, *layers[0])
    (s_all,) = _sc_pass(False)(xcat, srcs, dsts, z_c)
    xcat = _tc_pass(s_all, cnt_all, xcat, *layers[1])
    return xcat[:N], xcat[N:]


# speedup vs baseline: 3.0858x; 3.0858x over previous
import jax
import jax.numpy as jnp
from jax import lax
from jax.experimental import pallas as pl
from jax.experimental.pallas import tpu as pltpu
from jax.experimental.pallas import tpu_sc as plsc

N = 10000
C = 128
E = 320000

NC = 2
NS = 16
B = 128
G = 16
CHUNKS = -(-E // (NS * B * G)) * G
GROUPS = CHUNKS // G
EPT = CHUNKS * B
EPAD = NS * EPT
NPAD = 10112
SLICE = NPAD // NS

BLK = 1000
NBLK = N // BLK

_MESH = plsc.VectorSubcoreMesh(
    core_axis_name="c", subcore_axis_name="s", num_cores=NC,
    num_subcores=NS)


def _rows_body(xcat, srcs, dsts, z_c, s_out, sidx, didx, rows, sem, acc):
    c = lax.axis_index("c")
    s = lax.axis_index("s")
    pltpu.sync_copy(z_c, acc.at[pl.ds(s * SLICE, SLICE)])
    plsc.subcore_barrier()

    def group(gi, carry):
        pltpu.sync_copy(srcs.at[c, s, pl.ds(gi * G, G)], sidx)
        pltpu.sync_copy(dsts.at[c, s, pl.ds(gi * G, G)], didx)

        def step(j, carry2):
            pltpu.async_copy(xcat.at[sidx.at[j]], rows, sem).wait()
            pltpu.sync_copy(rows, acc.at[didx.at[j]], add=True)
            return carry2

        return lax.fori_loop(0, G, step, carry)

    lax.fori_loop(0, GROUPS, group, 0)
    plsc.subcore_barrier()
    sl = pl.ds(s * SLICE, SLICE)
    pltpu.sync_copy(acc.at[sl], s_out.at[c, sl])


_sc_rows = pl.kernel(
    _rows_body,
    out_type=[jax.ShapeDtypeStruct((NC, NPAD, C), jnp.float32)],
    mesh=_MESH,
    scratch_types=[
        pltpu.VMEM((G, B), jnp.int32),
        pltpu.VMEM((G, B), jnp.int32),
        pltpu.VMEM((B, C), jnp.float32),
        pltpu.SemaphoreType.DMA,
        pltpu.VMEM_SHARED((NPAD, C), jnp.float32),
    ])


def _cnt_body(dsts, z_c, ones_h, cnt_out, didx, ones_v, cacc):
    c = lax.axis_index("c")
    s = lax.axis_index("s")
    pltpu.sync_copy(z_c, cacc.at[pl.ds(s * SLICE, SLICE)])
    pltpu.sync_copy(ones_h, ones_v)
    plsc.subcore_barrier()

    def group(gi, carry):
        pltpu.sync_copy(dsts.at[c, s, pl.ds(gi * G, G)], didx)

        def step(j, carry2):
            pltpu.sync_copy(ones_v, cacc.at[didx.at[j]], add=True)
            return carry2

        return lax.fori_loop(0, G, step, carry)

    lax.fori_loop(0, GROUPS, group, 0)
    plsc.subcore_barrier()
    sl = pl.ds(s * SLICE, SLICE)
    pltpu.sync_copy(cacc.at[sl], cnt_out.at[c, sl])


_sc_cnt = pl.kernel(
    _cnt_body,
    out_type=[jax.ShapeDtypeStruct((NC, NPAD, C), jnp.float32)],
    mesh=_MESH,
    scratch_types=[
        pltpu.VMEM((G, B), jnp.int32),
        pltpu.VMEM((B, C), jnp.float32),
        pltpu.VMEM_SHARED((NPAD, C), jnp.float32),
    ])


def _tc_body(s_ref, cnt_ref, x_ref, wl_ref, bl_ref, wr_ref, lnw_ref,
             lnb_ref, o_ref):
    cnt = cnt_ref[0, :, 0:1]
    mean = s_ref[0] / jnp.maximum(cnt, 1.0)
    h = lax.dot_general(mean, wl_ref[0], (((1,), (1,)), ((), ())),
                        preferred_element_type=jnp.float32)
    h += lax.dot_general(x_ref[...], wr_ref[0], (((1,), (1,)), ((), ())),
                         preferred_element_type=jnp.float32)
    h += bl_ref[0]
    mu = jnp.mean(h, axis=1, keepdims=True)
    var = jnp.mean((h - mu) * (h - mu), axis=1, keepdims=True)
    y = (h - mu) * lax.rsqrt(var + 1e-5) * lnw_ref[0] + lnb_ref[0]
    o_ref[...] = jnp.maximum(y, 0.0)


_tc_pass = pl.pallas_call(
    _tc_body,
    grid=(2, NBLK),
    in_specs=[
        pl.BlockSpec((1, BLK, C), lambda t, i: (1 - t, i, 0)),
        pl.BlockSpec((1, BLK, C), lambda t, i: (1 - t, i, 0)),
        pl.BlockSpec((BLK, C), lambda t, i: (t * NBLK + i, 0)),
        pl.BlockSpec((1, C, C), lambda t, i: (t, 0, 0)),
        pl.BlockSpec((1, 1, C), lambda t, i: (t, 0, 0)),
        pl.BlockSpec((1, C, C), lambda t, i: (t, 0, 0)),
        pl.BlockSpec((1, 1, C), lambda t, i: (t, 0, 0)),
        pl.BlockSpec((1, 1, C), lambda t, i: (t, 0, 0)),
    ],
    out_specs=pl.BlockSpec((BLK, C), lambda t, i: (t * NBLK + i, 0)),
    out_shape=jax.ShapeDtypeStruct((2 * N, C), jnp.float32),
)


def _prep_edges(ei, src_off):
    pad = EPAD - E
    src = jnp.concatenate(
        [ei[0] + src_off, jnp.zeros((pad,), jnp.int32)])
    dst = jnp.concatenate(
        [ei[1], jnp.full((pad,), N, jnp.int32)])
    return src.reshape(NS, CHUNKS, B), dst.reshape(NS, CHUNKS, B)


def kernel(x_user, x_item, edge_index_user_item, edge_index_item_user,
           l0_ui_Wl, l0_ui_bl, l0_ui_Wr, l0_iu_Wl, l0_iu_bl, l0_iu_Wr,
           l0_ln_user_w, l0_ln_user_b, l0_ln_item_w, l0_ln_item_b,
           l1_ui_Wl, l1_ui_bl, l1_ui_Wr, l1_iu_Wl, l1_iu_bl, l1_iu_Wr,
           l1_ln_user_w, l1_ln_user_b, l1_ln_item_w, l1_ln_item_b):
    su, du = _prep_edges(edge_index_user_item, 0)
    si, di = _prep_edges(edge_index_item_user, N)
    srcs = jnp.stack([su, si])
    dsts = jnp.stack([du, di])
    z_c = jnp.zeros((SLICE, C), jnp.float32)
    ones_h = jnp.ones((B, C), jnp.float32)

    xcat = jnp.concatenate([x_user, x_item], axis=0)

    def _vec2(a, b):
        return jnp.stack([a, b])[:, None, :]

    layers = [
        (jnp.stack([l0_iu_Wl, l0_ui_Wl]), _vec2(l0_iu_bl, l0_ui_bl),
         jnp.stack([l0_iu_Wr, l0_ui_Wr]),
         _vec2(l0_ln_user_w, l0_ln_item_w),
         _vec2(l0_ln_user_b, l0_ln_item_b)),
        (jnp.stack([l1_iu_Wl, l1_ui_Wl]), _vec2(l1_iu_bl, l1_ui_bl),
         jnp.stack([l1_iu_Wr, l1_ui_Wr]),
         _vec2(l1_ln_user_w, l1_ln_item_w),
         _vec2(l1_ln_user_b, l1_ln_item_b)),
    ]

    (cnt_all,) = _sc_cnt(dsts, z_c, ones_h)
    (s_all,) = _sc_rows(xcat, srcs, dsts, z_c)
    xcat = _tc_pass(s_all, cnt_all, xcat, *layers[0])
    (s_all,) = _sc_rows(xcat, srcs, dsts, z_c)
    xcat = _tc_pass(s_all, cnt_all, xcat, *layers[1])
    return xcat[:N], xcat[N:]


# speedup vs baseline: 3.2552x; 1.0549x over previous
import jax
import jax.numpy as jnp
from jax import lax
from jax.experimental import pallas as pl
from jax.experimental.pallas import tpu as pltpu
from jax.experimental.pallas import tpu_sc as plsc

N = 10000
C = 128
E = 320000

NC = 2
NS = 16
B = 128
G = 16
CHUNKS = -(-E // (NS * B * G)) * G
GROUPS = CHUNKS // G
EPT = CHUNKS * B
EPAD = NS * EPT
NPAD = 10112
SLICE = NPAD // NS

BLK = 1000
NBLK = N // BLK

_MESH = plsc.VectorSubcoreMesh(
    core_axis_name="c", subcore_axis_name="s", num_cores=NC,
    num_subcores=NS)


def _rows_body(xcat, srcs, dsts, z_c, s_out, sidx, didx, rows0, rows1,
               sem0, sem1, acc):
    c = lax.axis_index("c")
    s = lax.axis_index("s")
    pltpu.sync_copy(z_c, acc.at[pl.ds(s * SLICE, SLICE)])
    plsc.subcore_barrier()
    rows = (rows0, rows1)
    sems = (sem0, sem1)

    def group(gi, carry):
        pltpu.sync_copy(srcs.at[c, s, pl.ds(gi * G, G)], sidx)
        pltpu.sync_copy(dsts.at[c, s, pl.ds(gi * G, G)], didx)
        pltpu.async_copy(xcat.at[sidx.at[0]], rows0, sem0)
        for j in range(G):
            b = j % 2
            pltpu.make_async_copy(xcat.at[sidx.at[j]], rows[b],
                                  sems[b]).wait()
            if j + 1 < G:
                pltpu.async_copy(xcat.at[sidx.at[j + 1]], rows[1 - b],
                                 sems[1 - b])
            pltpu.sync_copy(rows[b], acc.at[didx.at[j]], add=True)
        return carry

    lax.fori_loop(0, GROUPS, group, 0)
    plsc.subcore_barrier()
    sl = pl.ds(s * SLICE, SLICE)
    pltpu.sync_copy(acc.at[sl], s_out.at[c, sl])


_sc_rows = pl.kernel(
    _rows_body,
    out_type=[jax.ShapeDtypeStruct((NC, NPAD, C), jnp.float32)],
    mesh=_MESH,
    scratch_types=[
        pltpu.VMEM((G, B), jnp.int32),
        pltpu.VMEM((G, B), jnp.int32),
        pltpu.VMEM((B, C), jnp.float32),
        pltpu.VMEM((B, C), jnp.float32),
        pltpu.SemaphoreType.DMA,
        pltpu.SemaphoreType.DMA,
        pltpu.VMEM_SHARED((NPAD, C), jnp.float32),
    ])


def _cnt_body(dsts, z_c, ones_h, cnt_out, didx, ones_v, cacc):
    c = lax.axis_index("c")
    s = lax.axis_index("s")
    pltpu.sync_copy(z_c, cacc.at[pl.ds(s * SLICE, SLICE)])
    pltpu.sync_copy(ones_h, ones_v)
    plsc.subcore_barrier()

    def group(gi, carry):
        pltpu.sync_copy(dsts.at[c, s, pl.ds(gi * G, G)], didx)

        def step(j, carry2):
            pltpu.sync_copy(ones_v, cacc.at[didx.at[j]], add=True)
            return carry2

        return lax.fori_loop(0, G, step, carry)

    lax.fori_loop(0, GROUPS, group, 0)
    plsc.subcore_barrier()
    sl = pl.ds(s * SLICE, SLICE)
    pltpu.sync_copy(cacc.at[sl], cnt_out.at[c, sl])


_sc_cnt = pl.kernel(
    _cnt_body,
    out_type=[jax.ShapeDtypeStruct((NC, NPAD, C), jnp.float32)],
    mesh=_MESH,
    scratch_types=[
        pltpu.VMEM((G, B), jnp.int32),
        pltpu.VMEM((B, C), jnp.float32),
        pltpu.VMEM_SHARED((NPAD, C), jnp.float32),
    ])


def _tc_body(s_ref, cnt_ref, x_ref, wl_ref, bl_ref, wr_ref, lnw_ref,
             lnb_ref, o_ref):
    cnt = cnt_ref[0, :, 0:1]
    mean = s_ref[0] / jnp.maximum(cnt, 1.0)
    h = lax.dot_general(mean, wl_ref[0], (((1,), (1,)), ((), ())),
                        preferred_element_type=jnp.float32)
    h += lax.dot_general(x_ref[...], wr_ref[0], (((1,), (1,)), ((), ())),
                         preferred_element_type=jnp.float32)
    h += bl_ref[0]
    mu = jnp.mean(h, axis=1, keepdims=True)
    var = jnp.mean((h - mu) * (h - mu), axis=1, keepdims=True)
    y = (h - mu) * lax.rsqrt(var + 1e-5) * lnw_ref[0] + lnb_ref[0]
    o_ref[...] = jnp.maximum(y, 0.0)


_tc_pass = pl.pallas_call(
    _tc_body,
    grid=(2, NBLK),
    in_specs=[
        pl.BlockSpec((1, BLK, C), lambda t, i: (1 - t, i, 0)),
        pl.BlockSpec((1, BLK, C), lambda t, i: (1 - t, i, 0)),
        pl.BlockSpec((BLK, C), lambda t, i: (t * NBLK + i, 0)),
        pl.BlockSpec((1, C, C), lambda t, i: (t, 0, 0)),
        pl.BlockSpec((1, 1, C), lambda t, i: (t, 0, 0)),
        pl.BlockSpec((1, C, C), lambda t, i: (t, 0, 0)),
        pl.BlockSpec((1, 1, C), lambda t, i: (t, 0, 0)),
        pl.BlockSpec((1, 1, C), lambda t, i: (t, 0, 0)),
    ],
    out_specs=pl.BlockSpec((BLK, C), lambda t, i: (t * NBLK + i, 0)),
    out_shape=jax.ShapeDtypeStruct((2 * N, C), jnp.float32),
)


def _prep_edges(ei, src_off):
    pad = EPAD - E
    src = jnp.concatenate(
        [ei[0] + src_off, jnp.zeros((pad,), jnp.int32)])
    dst = jnp.concatenate(
        [ei[1], jnp.full((pad,), N, jnp.int32)])
    return src.reshape(NS, CHUNKS, B), dst.reshape(NS, CHUNKS, B)


def kernel(x_user, x_item, edge_index_user_item, edge_index_item_user,
           l0_ui_Wl, l0_ui_bl, l0_ui_Wr, l0_iu_Wl, l0_iu_bl, l0_iu_Wr,
           l0_ln_user_w, l0_ln_user_b, l0_ln_item_w, l0_ln_item_b,
           l1_ui_Wl, l1_ui_bl, l1_ui_Wr, l1_iu_Wl, l1_iu_bl, l1_iu_Wr,
           l1_ln_user_w, l1_ln_user_b, l1_ln_item_w, l1_ln_item_b):
    su, du = _prep_edges(edge_index_user_item, 0)
    si, di = _prep_edges(edge_index_item_user, N)
    srcs = jnp.stack([su, si])
    dsts = jnp.stack([du, di])
    z_c = jnp.zeros((SLICE, C), jnp.float32)
    ones_h = jnp.ones((B, C), jnp.float32)

    xcat = jnp.concatenate([x_user, x_item], axis=0)

    def _vec2(a, b):
        return jnp.stack([a, b])[:, None, :]

    layers = [
        (jnp.stack([l0_iu_Wl, l0_ui_Wl]), _vec2(l0_iu_bl, l0_ui_bl),
         jnp.stack([l0_iu_Wr, l0_ui_Wr]),
         _vec2(l0_ln_user_w, l0_ln_item_w),
         _vec2(l0_ln_user_b, l0_ln_item_b)),
        (jnp.stack([l1_iu_Wl, l1_ui_Wl]), _vec2(l1_iu_bl, l1_ui_bl),
         jnp.stack([l1_iu_Wr, l1_ui_Wr]),
         _vec2(l1_ln_user_w, l1_ln_item_w),
         _vec2(l1_ln_user_b, l1_ln_item_b)),
    ]

    (cnt_all,) = _sc_cnt(dsts, z_c, ones_h)
    (s_all,) = _sc_rows(xcat, srcs, dsts, z_c)
    xcat = _tc_pass(s_all, cnt_all, xcat, *layers[0])
    (s_all,) = _sc_rows(xcat, srcs, dsts, z_c)
    xcat = _tc_pass(s_all, cnt_all, xcat, *layers[1])
    return xcat[:N], xcat[N:]


# speedup vs baseline: 3.3422x; 1.0267x over previous
import jax
import jax.numpy as jnp
from jax import lax
from jax.experimental import pallas as pl
from jax.experimental.pallas import tpu as pltpu
from jax.experimental.pallas import tpu_sc as plsc

N = 10000
C = 128
E = 320000

NC = 2
NS = 16
B = 128
G = 16
CHUNKS = -(-E // (NS * B * G)) * G
GROUPS = CHUNKS // G
EPT = CHUNKS * B
EPAD = NS * EPT
NPAD = 10112
SLICE = NPAD // NS

BLK = 1000
NBLK = N // BLK

_MESH = plsc.VectorSubcoreMesh(
    core_axis_name="c", subcore_axis_name="s", num_cores=NC,
    num_subcores=NS)


def _rows_body(xcat, srcs, dsts, z_c, s_out, sidx, didx, rows0, rows1,
               sem0, sem1, acc):
    c = lax.axis_index("c")
    s = lax.axis_index("s")
    pltpu.sync_copy(z_c, acc.at[pl.ds(s * SLICE, SLICE)])
    plsc.subcore_barrier()
    rows = (rows0, rows1)
    sems = (sem0, sem1)

    def group(gi, carry):
        pltpu.sync_copy(srcs.at[c, s, pl.ds(gi * G, G)], sidx)
        pltpu.sync_copy(dsts.at[c, s, pl.ds(gi * G, G)], didx)
        pltpu.async_copy(xcat.at[sidx.at[0]], rows0, sem0)
        pltpu.async_copy(xcat.at[sidx.at[1]], rows1, sem1)
        for j in range(G):
            b = j % 2
            pltpu.make_async_copy(xcat.at[sidx.at[j]], rows[b],
                                  sems[b]).wait()
            pltpu.sync_copy(rows[b], acc.at[didx.at[j]], add=True)
            if j + 2 < G:
                pltpu.async_copy(xcat.at[sidx.at[j + 2]], rows[b],
                                 sems[b])
        return carry

    lax.fori_loop(0, GROUPS, group, 0)
    plsc.subcore_barrier()
    sl = pl.ds(s * SLICE, SLICE)
    pltpu.sync_copy(acc.at[sl], s_out.at[c, sl])


_sc_rows = pl.kernel(
    _rows_body,
    out_type=[jax.ShapeDtypeStruct((NC, NPAD, C), jnp.float32)],
    mesh=_MESH,
    scratch_types=[
        pltpu.VMEM((G, B), jnp.int32),
        pltpu.VMEM((G, B), jnp.int32),
        pltpu.VMEM((B, C), jnp.float32),
        pltpu.VMEM((B, C), jnp.float32),
        pltpu.SemaphoreType.DMA,
        pltpu.SemaphoreType.DMA,
        pltpu.VMEM_SHARED((NPAD, C), jnp.float32),
    ])


def _cnt_body(dsts, z_c, ones_h, cnt_out, didx, ones_v, cacc):
    c = lax.axis_index("c")
    s = lax.axis_index("s")
    pltpu.sync_copy(z_c, cacc.at[pl.ds(s * SLICE, SLICE)])
    pltpu.sync_copy(ones_h, ones_v)
    plsc.subcore_barrier()

    def group(gi, carry):
        pltpu.sync_copy(dsts.at[c, s, pl.ds(gi * G, G)], didx)

        def step(j, carry2):
            pltpu.sync_copy(ones_v, cacc.at[didx.at[j]], add=True)
            return carry2

        return lax.fori_loop(0, G, step, carry)

    lax.fori_loop(0, GROUPS, group, 0)
    plsc.subcore_barrier()
    sl = pl.ds(s * SLICE, SLICE)
    pltpu.sync_copy(cacc.at[sl], cnt_out.at[c, sl])


_sc_cnt = pl.kernel(
    _cnt_body,
    out_type=[jax.ShapeDtypeStruct((NC, NPAD, C), jnp.float32)],
    mesh=_MESH,
    scratch_types=[
        pltpu.VMEM((G, B), jnp.int32),
        pltpu.VMEM((B, C), jnp.float32),
        pltpu.VMEM_SHARED((NPAD, C), jnp.float32),
    ])


def _tc_body(s_ref, cnt_ref, x_ref, wl_ref, bl_ref, wr_ref, lnw_ref,
             lnb_ref, o_ref):
    cnt = cnt_ref[0, :, 0:1]
    mean = s_ref[0] / jnp.maximum(cnt, 1.0)
    h = lax.dot_general(mean, wl_ref[0], (((1,), (1,)), ((), ())),
                        preferred_element_type=jnp.float32)
    h += lax.dot_general(x_ref[...], wr_ref[0], (((1,), (1,)), ((), ())),
                         preferred_element_type=jnp.float32)
    h += bl_ref[0]
    mu = jnp.mean(h, axis=1, keepdims=True)
    var = jnp.mean((h - mu) * (h - mu), axis=1, keepdims=True)
    y = (h - mu) * lax.rsqrt(var + 1e-5) * lnw_ref[0] + lnb_ref[0]
    o_ref[...] = jnp.maximum(y, 0.0)


_tc_pass = pl.pallas_call(
    _tc_body,
    grid=(2, NBLK),
    in_specs=[
        pl.BlockSpec((1, BLK, C), lambda t, i: (1 - t, i, 0)),
        pl.BlockSpec((1, BLK, C), lambda t, i: (1 - t, i, 0)),
        pl.BlockSpec((BLK, C), lambda t, i: (t * NBLK + i, 0)),
        pl.BlockSpec((1, C, C), lambda t, i: (t, 0, 0)),
        pl.BlockSpec((1, 1, C), lambda t, i: (t, 0, 0)),
        pl.BlockSpec((1, C, C), lambda t, i: (t, 0, 0)),
        pl.BlockSpec((1, 1, C), lambda t, i: (t, 0, 0)),
        pl.BlockSpec((1, 1, C), lambda t, i: (t, 0, 0)),
    ],
    out_specs=pl.BlockSpec((BLK, C), lambda t, i: (t * NBLK + i, 0)),
    out_shape=jax.ShapeDtypeStruct((2 * N, C), jnp.float32),
)


def _prep_edges(ei, src_off):
    pad = EPAD - E
    src = jnp.concatenate(
        [ei[0] + src_off, jnp.zeros((pad,), jnp.int32)])
    dst = jnp.concatenate(
        [ei[1], jnp.full((pad,), N, jnp.int32)])
    return src.reshape(NS, CHUNKS, B), dst.reshape(NS, CHUNKS, B)


def kernel(x_user, x_item, edge_index_user_item, edge_index_item_user,
           l0_ui_Wl, l0_ui_bl, l0_ui_Wr, l0_iu_Wl, l0_iu_bl, l0_iu_Wr,
           l0_ln_user_w, l0_ln_user_b, l0_ln_item_w, l0_ln_item_b,
           l1_ui_Wl, l1_ui_bl, l1_ui_Wr, l1_iu_Wl, l1_iu_bl, l1_iu_Wr,
           l1_ln_user_w, l1_ln_user_b, l1_ln_item_w, l1_ln_item_b):
    su, du = _prep_edges(edge_index_user_item, 0)
    si, di = _prep_edges(edge_index_item_user, N)
    srcs = jnp.stack([su, si])
    dsts = jnp.stack([du, di])
    z_c = jnp.zeros((SLICE, C), jnp.float32)
    ones_h = jnp.ones((B, C), jnp.float32)

    xcat = jnp.concatenate([x_user, x_item], axis=0)

    def _vec2(a, b):
        return jnp.stack([a, b])[:, None, :]

    layers = [
        (jnp.stack([l0_iu_Wl, l0_ui_Wl]), _vec2(l0_iu_bl, l0_ui_bl),
         jnp.stack([l0_iu_Wr, l0_ui_Wr]),
         _vec2(l0_ln_user_w, l0_ln_item_w),
         _vec2(l0_ln_user_b, l0_ln_item_b)),
        (jnp.stack([l1_iu_Wl, l1_ui_Wl]), _vec2(l1_iu_bl, l1_ui_bl),
         jnp.stack([l1_iu_Wr, l1_ui_Wr]),
         _vec2(l1_ln_user_w, l1_ln_item_w),
         _vec2(l1_ln_user_b, l1_ln_item_b)),
    ]

    (cnt_all,) = _sc_cnt(dsts, z_c, ones_h)
    (s_all,) = _sc_rows(xcat, srcs, dsts, z_c)
    xcat = _tc_pass(s_all, cnt_all, xcat, *layers[0])
    (s_all,) = _sc_rows(xcat, srcs, dsts, z_c)
    xcat = _tc_pass(s_all, cnt_all, xcat, *layers[1])
    return xcat[:N], xcat[N:]


# speedup vs baseline: 3.3992x; 1.0171x over previous
import jax
import jax.numpy as jnp
from jax import lax
from jax.experimental import pallas as pl
from jax.experimental.pallas import tpu as pltpu
from jax.experimental.pallas import tpu_sc as plsc

N = 10000
C = 128
E = 320000

NC = 2
NS = 16
B = 128
G = 32
CHUNKS = -(-E // (NS * B * G)) * G
GROUPS = CHUNKS // G
EPT = CHUNKS * B
EPAD = NS * EPT
NPAD = 10112
SLICE = NPAD // NS

BLK = 2000
NBLK = N // BLK

_MESH = plsc.VectorSubcoreMesh(
    core_axis_name="c", subcore_axis_name="s", num_cores=NC,
    num_subcores=NS)


def _rows_body(xcat, srcs, dsts, z_c, s_out, sidx, didx, rows0, rows1,
               sem0, sem1, acc):
    c = lax.axis_index("c")
    s = lax.axis_index("s")
    pltpu.sync_copy(z_c, acc.at[pl.ds(s * SLICE, SLICE)])
    plsc.subcore_barrier()
    rows = (rows0, rows1)
    sems = (sem0, sem1)

    def group(gi, carry):
        pltpu.sync_copy(srcs.at[c, s, pl.ds(gi * G, G)], sidx)
        pltpu.sync_copy(dsts.at[c, s, pl.ds(gi * G, G)], didx)
        pltpu.async_copy(xcat.at[sidx.at[0]], rows0, sem0)
        pltpu.async_copy(xcat.at[sidx.at[1]], rows1, sem1)
        for j in range(G):
            b = j % 2
            pltpu.make_async_copy(xcat.at[sidx.at[j]], rows[b],
                                  sems[b]).wait()
            pltpu.sync_copy(rows[b], acc.at[didx.at[j]], add=True)
            if j + 2 < G:
                pltpu.async_copy(xcat.at[sidx.at[j + 2]], rows[b],
                                 sems[b])
        return carry

    lax.fori_loop(0, GROUPS, group, 0)
    plsc.subcore_barrier()
    sl = pl.ds(s * SLICE, SLICE)
    pltpu.sync_copy(acc.at[sl], s_out.at[c, sl])


_sc_rows = pl.kernel(
    _rows_body,
    out_type=[jax.ShapeDtypeStruct((NC, NPAD, C), jnp.float32)],
    mesh=_MESH,
    scratch_types=[
        pltpu.VMEM((G, B), jnp.int32),
        pltpu.VMEM((G, B), jnp.int32),
        pltpu.VMEM((B, C), jnp.float32),
        pltpu.VMEM((B, C), jnp.float32),
        pltpu.SemaphoreType.DMA,
        pltpu.SemaphoreType.DMA,
        pltpu.VMEM_SHARED((NPAD, C), jnp.float32),
    ])


def _cnt_body(dsts, z_c, ones_h, cnt_out, didx, ones_v, cacc):
    c = lax.axis_index("c")
    s = lax.axis_index("s")
    pltpu.sync_copy(z_c, cacc.at[pl.ds(s * SLICE, SLICE)])
    pltpu.sync_copy(ones_h, ones_v)
    plsc.subcore_barrier()

    def group(gi, carry):
        pltpu.sync_copy(dsts.at[c, s, pl.ds(gi * G, G)], didx)

        def step(j, carry2):
            pltpu.sync_copy(ones_v, cacc.at[didx.at[j]], add=True)
            return carry2

        return lax.fori_loop(0, G, step, carry)

    lax.fori_loop(0, GROUPS, group, 0)
    plsc.subcore_barrier()
    sl = pl.ds(s * SLICE, SLICE)
    pltpu.sync_copy(cacc.at[sl], cnt_out.at[c, sl])


_sc_cnt = pl.kernel(
    _cnt_body,
    out_type=[jax.ShapeDtypeStruct((NC, NPAD, C), jnp.float32)],
    mesh=_MESH,
    scratch_types=[
        pltpu.VMEM((G, B), jnp.int32),
        pltpu.VMEM((B, C), jnp.float32),
        pltpu.VMEM_SHARED((NPAD, C), jnp.float32),
    ])


def _tc_body(s_ref, cnt_ref, x_ref, wl_ref, bl_ref, wr_ref, lnw_ref,
             lnb_ref, o_ref):
    cnt = cnt_ref[0, :, 0:1]
    mean = s_ref[0] / jnp.maximum(cnt, 1.0)
    h = lax.dot_general(mean, wl_ref[0], (((1,), (1,)), ((), ())),
                        preferred_element_type=jnp.float32)
    h += lax.dot_general(x_ref[...], wr_ref[0], (((1,), (1,)), ((), ())),
                         preferred_element_type=jnp.float32)
    h += bl_ref[0]
    mu = jnp.mean(h, axis=1, keepdims=True)
    var = jnp.mean((h - mu) * (h - mu), axis=1, keepdims=True)
    y = (h - mu) * lax.rsqrt(var + 1e-5) * lnw_ref[0] + lnb_ref[0]
    o_ref[...] = jnp.maximum(y, 0.0)


_tc_pass = pl.pallas_call(
    _tc_body,
    grid=(2, NBLK),
    in_specs=[
        pl.BlockSpec((1, BLK, C), lambda t, i: (1 - t, i, 0)),
        pl.BlockSpec((1, BLK, C), lambda t, i: (1 - t, i, 0)),
        pl.BlockSpec((BLK, C), lambda t, i: (t * NBLK + i, 0)),
        pl.BlockSpec((1, C, C), lambda t, i: (t, 0, 0)),
        pl.BlockSpec((1, 1, C), lambda t, i: (t, 0, 0)),
        pl.BlockSpec((1, C, C), lambda t, i: (t, 0, 0)),
        pl.BlockSpec((1, 1, C), lambda t, i: (t, 0, 0)),
        pl.BlockSpec((1, 1, C), lambda t, i: (t, 0, 0)),
    ],
    out_specs=pl.BlockSpec((BLK, C), lambda t, i: (t * NBLK + i, 0)),
    out_shape=jax.ShapeDtypeStruct((2 * N, C), jnp.float32),
)


def _prep_edges(ei, src_off):
    pad = EPAD - E
    src = jnp.concatenate(
        [ei[0] + src_off, jnp.zeros((pad,), jnp.int32)])
    dst = jnp.concatenate(
        [ei[1], jnp.full((pad,), N, jnp.int32)])
    return src.reshape(NS, CHUNKS, B), dst.reshape(NS, CHUNKS, B)


def kernel(x_user, x_item, edge_index_user_item, edge_index_item_user,
           l0_ui_Wl, l0_ui_bl, l0_ui_Wr, l0_iu_Wl, l0_iu_bl, l0_iu_Wr,
           l0_ln_user_w, l0_ln_user_b, l0_ln_item_w, l0_ln_item_b,
           l1_ui_Wl, l1_ui_bl, l1_ui_Wr, l1_iu_Wl, l1_iu_bl, l1_iu_Wr,
           l1_ln_user_w, l1_ln_user_b, l1_ln_item_w, l1_ln_item_b):
    su, du = _prep_edges(edge_index_user_item, 0)
    si, di = _prep_edges(edge_index_item_user, N)
    srcs = jnp.stack([su, si])
    dsts = jnp.stack([du, di])
    z_c = jnp.zeros((SLICE, C), jnp.float32)
    ones_h = jnp.ones((B, C), jnp.float32)

    xcat = jnp.concatenate([x_user, x_item], axis=0)

    def _vec2(a, b):
        return jnp.stack([a, b])[:, None, :]

    layers = [
        (jnp.stack([l0_iu_Wl, l0_ui_Wl]), _vec2(l0_iu_bl, l0_ui_bl),
         jnp.stack([l0_iu_Wr, l0_ui_Wr]),
         _vec2(l0_ln_user_w, l0_ln_item_w),
         _vec2(l0_ln_user_b, l0_ln_item_b)),
        (jnp.stack([l1_iu_Wl, l1_ui_Wl]), _vec2(l1_iu_bl, l1_ui_bl),
         jnp.stack([l1_iu_Wr, l1_ui_Wr]),
         _vec2(l1_ln_user_w, l1_ln_item_w),
         _vec2(l1_ln_user_b, l1_ln_item_b)),
    ]

    (cnt_all,) = _sc_cnt(dsts, z_c, ones_h)
    (s_all,) = _sc_rows(xcat, srcs, dsts, z_c)
    xcat = _tc_pass(s_all, cnt_all, xcat, *layers[0])
    (s_all,) = _sc_rows(xcat, srcs, dsts, z_c)
    xcat = _tc_pass(s_all, cnt_all, xcat, *layers[1])
    return xcat[:N], xcat[N:]


# speedup vs baseline: 3.4112x; 1.0035x over previous
import jax
import jax.numpy as jnp
from jax import lax
from jax.experimental import pallas as pl
from jax.experimental.pallas import tpu as pltpu
from jax.experimental.pallas import tpu_sc as plsc

N = 10000
C = 128
E = 320000

NC = 2
NS = 16
B = 128
G = 32
CHUNKS = -(-E // (NS * B * G)) * G
GROUPS = CHUNKS // G
EPT = CHUNKS * B
EPAD = NS * EPT
NPAD = 10112
SLICE = NPAD // NS

BLK = 2000
NBLK = N // BLK

_MESH = plsc.VectorSubcoreMesh(
    core_axis_name="c", subcore_axis_name="s", num_cores=NC,
    num_subcores=NS)


def _rows_loop(xcat, srcs, dsts, sidx, didx, rows0, rows1, sem0, sem1,
               acc, c, s):
    rows = (rows0, rows1)
    sems = (sem0, sem1)

    def group(gi, carry):
        pltpu.sync_copy(srcs.at[c, s, pl.ds(gi * G, G)], sidx)
        pltpu.sync_copy(dsts.at[c, s, pl.ds(gi * G, G)], didx)
        pltpu.async_copy(xcat.at[sidx.at[0]], rows0, sem0)
        pltpu.async_copy(xcat.at[sidx.at[1]], rows1, sem1)
        for j in range(G):
            b = j % 2
            pltpu.make_async_copy(xcat.at[sidx.at[j]], rows[b],
                                  sems[b]).wait()
            pltpu.sync_copy(rows[b], acc.at[didx.at[j]], add=True)
            if j + 2 < G:
                pltpu.async_copy(xcat.at[sidx.at[j + 2]], rows[b],
                                 sems[b])
        return carry

    lax.fori_loop(0, GROUPS, group, 0)


def _rows_body(xcat, srcs, dsts, z_c, s_out, sidx, didx, rows0, rows1,
               sem0, sem1, acc):
    c = lax.axis_index("c")
    s = lax.axis_index("s")
    pltpu.sync_copy(z_c, acc.at[pl.ds(s * SLICE, SLICE)])
    plsc.subcore_barrier()
    _rows_loop(xcat, srcs, dsts, sidx, didx, rows0, rows1, sem0, sem1,
               acc, c, s)
    plsc.subcore_barrier()
    sl = pl.ds(s * SLICE, SLICE)
    pltpu.sync_copy(acc.at[sl], s_out.at[c, sl])


_sc_rows = pl.kernel(
    _rows_body,
    out_type=[jax.ShapeDtypeStruct((NC, NPAD, C), jnp.float32)],
    mesh=_MESH,
    scratch_types=[
        pltpu.VMEM((G, B), jnp.int32),
        pltpu.VMEM((G, B), jnp.int32),
        pltpu.VMEM((B, C), jnp.float32),
        pltpu.VMEM((B, C), jnp.float32),
        pltpu.SemaphoreType.DMA,
        pltpu.SemaphoreType.DMA,
        pltpu.VMEM_SHARED((NPAD, C), jnp.float32),
    ])


def _rows_cnt_body(xcat, srcs, dsts, z_c, ones_h, s_out, cnt_out,
                   sidx, didx, rows0, rows1, sem0, sem1, acc):
    c = lax.axis_index("c")
    s = lax.axis_index("s")
    sl = pl.ds(s * SLICE, SLICE)
    pltpu.sync_copy(z_c, acc.at[sl])
    pltpu.sync_copy(ones_h, rows0)
    plsc.subcore_barrier()

    def cgroup(gi, carry):
        pltpu.sync_copy(dsts.at[c, s, pl.ds(gi * G, G)], didx)

        def step(j, carry2):
            pltpu.sync_copy(rows0, acc.at[didx.at[j]], add=True)
            return carry2

        return lax.fori_loop(0, G, step, carry)

    lax.fori_loop(0, GROUPS, cgroup, 0)
    plsc.subcore_barrier()
    pltpu.sync_copy(acc.at[sl], cnt_out.at[c, sl])
    pltpu.sync_copy(z_c, acc.at[sl])
    plsc.subcore_barrier()
    _rows_loop(xcat, srcs, dsts, sidx, didx, rows0, rows1, sem0, sem1,
               acc, c, s)
    plsc.subcore_barrier()
    pltpu.sync_copy(acc.at[sl], s_out.at[c, sl])


_sc_rows_cnt = pl.kernel(
    _rows_cnt_body,
    out_type=[jax.ShapeDtypeStruct((NC, NPAD, C), jnp.float32),
              jax.ShapeDtypeStruct((NC, NPAD, C), jnp.float32)],
    mesh=_MESH,
    scratch_types=[
        pltpu.VMEM((G, B), jnp.int32),
        pltpu.VMEM((G, B), jnp.int32),
        pltpu.VMEM((B, C), jnp.float32),
        pltpu.VMEM((B, C), jnp.float32),
        pltpu.SemaphoreType.DMA,
        pltpu.SemaphoreType.DMA,
        pltpu.VMEM_SHARED((NPAD, C), jnp.float32),
    ])


def _tc_body(s_ref, cnt_ref, x_ref, wl_ref, bl_ref, wr_ref, lnw_ref,
             lnb_ref, o_ref):
    cnt = cnt_ref[0, :, 0:1]
    mean = s_ref[0] / jnp.maximum(cnt, 1.0)
    h = lax.dot_general(mean, wl_ref[0], (((1,), (1,)), ((), ())),
                        preferred_element_type=jnp.float32)
    h += lax.dot_general(x_ref[...], wr_ref[0], (((1,), (1,)), ((), ())),
                         preferred_element_type=jnp.float32)
    h += bl_ref[0]
    mu = jnp.mean(h, axis=1, keepdims=True)
    var = jnp.mean((h - mu) * (h - mu), axis=1, keepdims=True)
    y = (h - mu) * lax.rsqrt(var + 1e-5) * lnw_ref[0] + lnb_ref[0]
    o_ref[...] = jnp.maximum(y, 0.0)


_tc_pass = pl.pallas_call(
    _tc_body,
    grid=(2, NBLK),
    in_specs=[
        pl.BlockSpec((1, BLK, C), lambda t, i: (1 - t, i, 0)),
        pl.BlockSpec((1, BLK, C), lambda t, i: (1 - t, i, 0)),
        pl.BlockSpec((BLK, C), lambda t, i: (t * NBLK + i, 0)),
        pl.BlockSpec((1, C, C), lambda t, i: (t, 0, 0)),
        pl.BlockSpec((1, 1, C), lambda t, i: (t, 0, 0)),
        pl.BlockSpec((1, C, C), lambda t, i: (t, 0, 0)),
        pl.BlockSpec((1, 1, C), lambda t, i: (t, 0, 0)),
        pl.BlockSpec((1, 1, C), lambda t, i: (t, 0, 0)),
    ],
    out_specs=pl.BlockSpec((BLK, C), lambda t, i: (t * NBLK + i, 0)),
    out_shape=jax.ShapeDtypeStruct((2 * N, C), jnp.float32),
)


def _prep_edges(ei, src_off):
    pad = EPAD - E
    src = jnp.concatenate(
        [ei[0] + src_off, jnp.zeros((pad,), jnp.int32)])
    dst = jnp.concatenate(
        [ei[1], jnp.full((pad,), N, jnp.int32)])
    return src.reshape(NS, CHUNKS, B), dst.reshape(NS, CHUNKS, B)


def kernel(x_user, x_item, edge_index_user_item, edge_index_item_user,
           l0_ui_Wl, l0_ui_bl, l0_ui_Wr, l0_iu_Wl, l0_iu_bl, l0_iu_Wr,
           l0_ln_user_w, l0_ln_user_b, l0_ln_item_w, l0_ln_item_b,
           l1_ui_Wl, l1_ui_bl, l1_ui_Wr, l1_iu_Wl, l1_iu_bl, l1_iu_Wr,
           l1_ln_user_w, l1_ln_user_b, l1_ln_item_w, l1_ln_item_b):
    su, du = _prep_edges(edge_index_user_item, 0)
    si, di = _prep_edges(edge_index_item_user, N)
    srcs = jnp.stack([su, si])
    dsts = jnp.stack([du, di])
    z_c = jnp.zeros((SLICE, C), jnp.float32)
    ones_h = jnp.ones((B, C), jnp.float32)

    xcat = jnp.concatenate([x_user, x_item], axis=0)

    def _vec2(a, b):
        return jnp.stack([a, b])[:, None, :]

    layers = [
        (jnp.stack([l0_iu_Wl, l0_ui_Wl]), _vec2(l0_iu_bl, l0_ui_bl),
         jnp.stack([l0_iu_Wr, l0_ui_Wr]),
         _vec2(l0_ln_user_w, l0_ln_item_w),
         _vec2(l0_ln_user_b, l0_ln_item_b)),
        (jnp.stack([l1_iu_Wl, l1_ui_Wl]), _vec2(l1_iu_bl, l1_ui_bl),
         jnp.stack([l1_iu_Wr, l1_ui_Wr]),
         _vec2(l1_ln_user_w, l1_ln_item_w),
         _vec2(l1_ln_user_b, l1_ln_item_b)),
    ]

    s_all, cnt_all = _sc_rows_cnt(xcat, srcs, dsts, z_c, ones_h)
    xcat = _tc_pass(s_all, cnt_all, xcat, *layers[0])
    (s_all,) = _sc_rows(xcat, srcs, dsts, z_c)
    xcat = _tc_pass(s_all, cnt_all, xcat, *layers[1])
    return xcat[:N], xcat[N:]
